# baseline (speedup 1.0000x reference)
import functools

import jax
import jax.numpy as jnp
from jax import lax
from jax.experimental import pallas as pl
from jax.experimental.pallas import tpu as pltpu
from jax.experimental.pallas import tpu_sc as plsc

_EMB = 64
_B = 16384
_L = 20
_NC = 2
_NS = 16
_NW = _NC * _NS
_BPW = _B // _NW
_CB = 32
_NSUB = _BPW // _CB
_ROWS = _CB * _L
_GCH = 128
_NG = _ROWS // _GCH
_DC = 8

_mesh = plsc.VectorSubcoreMesh(
    core_axis_name="c", subcore_axis_name="s", num_cores=_NC, num_subcores=_NS
)


@functools.partial(
    pl.kernel,
    out_type=jax.ShapeDtypeStruct((_B * _L,), jnp.float32),
    mesh=_mesh,
    compiler_params=pltpu.CompilerParams(
        use_tc_tiling_on_sc=False, needs_layout_passes=False),
    scratch_types=[
        pltpu.VMEM((_BPW,), jnp.int32),
        pltpu.VMEM((_BPW * _L,), jnp.int32),
        pltpu.VMEM((2 * _CB, _EMB), jnp.float32),
        pltpu.VMEM((2 * _ROWS, _EMB), jnp.float32),
        pltpu.VMEM((_BPW * _L,), jnp.float32),
        pltpu.SemaphoreType.DMA,
    ],
)
def _sc_skipgram(central_hbm, ctxidx_hbm, win_hbm, wout_hbm, out_hbm,
                 cidx_v, xidx_v, hid_v, ctx_v, out_v, sem):
    wid = lax.axis_index("s") * _NC + lax.axis_index("c")
    base = wid * _BPW
    pltpu.sync_copy(central_hbm.at[pl.ds(base, _BPW)], cidx_v)
    pltpu.sync_copy(ctxidx_hbm.at[pl.ds(base * _L, _BPW * _L)], xidx_v)

    lanes = lax.iota(jnp.int32, 16)

    def fire(sc, hofs, cofs):
        pltpu.async_copy(
            win_hbm.at[cidx_v.at[pl.ds(sc * _CB, _CB)]],
            hid_v.at[pl.ds(hofs, _CB)], sem)
        for j in range(_NG):
            pltpu.async_copy(
                wout_hbm.at[xidx_v.at[pl.ds(sc * _ROWS + j * _GCH, _GCH)]],
                ctx_v.at[pl.ds(cofs + j * _GCH, _GCH)], sem)

    fire(0, 0, 0)

    def sub(sc, _):
        p = sc % 2
        hofs = p * _CB
        cofs = p * _ROWS
        pltpu.make_async_copy(
            win_hbm.at[pl.ds(0, _CB)], hid_v.at[pl.ds(hofs, _CB)], sem).wait()
        pltpu.make_async_copy(
            wout_hbm.at[pl.ds(0, _ROWS)], ctx_v.at[pl.ds(cofs, _ROWS)],
            sem).wait()

        @pl.when(sc < _NSUB - 1)
        def _():
            q = (sc + 1) % 2
            fire(sc + 1, q * _CB, q * _ROWS)

        for g in range(_CB // 16):
            bvec = lanes + g * 16
            hrow = bvec + hofs
            crow = bvec * _L + cofs
            obase = sc * _ROWS + bvec * _L

            def dstep(dc, accs):
                d0 = dc * _DC
                dcols = [jnp.full((16,), d0 + dd, jnp.int32)
                         for dd in range(_DC)]
                hvs = [plsc.load_gather(hid_v, [hrow, dcols[dd]])
                       for dd in range(_DC)]
                out = []
                for l in range(_L):
                    rowv = crow + l
                    acc = accs[l]
                    for dd in range(_DC):
                        cv = plsc.load_gather(ctx_v, [rowv, dcols[dd]])
                        acc = acc + cv * hvs[dd]
                    out.append(acc)
                return tuple(out)

            accs = lax.fori_loop(
                0, _EMB // _DC, dstep,
                tuple(jnp.zeros(16, jnp.float32) for _ in range(_L)))

            for l in range(_L):
                sig = 1.0 / (1.0 + jnp.exp(-accs[l]))
                plsc.store_scatter(out_v, [obase + l], sig)
        return 0

    lax.fori_loop(0, _NSUB, sub, 0)
    pltpu.sync_copy(out_v, out_hbm.at[pl.ds(base * _L, _BPW * _L)])


def kernel(central_items, context_items, W_in, W_out):
    out = _sc_skipgram(
        central_items.astype(jnp.int32),
        context_items.reshape(-1).astype(jnp.int32),
        W_in,
        W_out,
    )
    return out.reshape(_B, _L)


# speedup vs baseline: 1.3675x; 1.3675x over previous
import functools

import jax
import jax.numpy as jnp
from jax import lax
from jax.experimental import pallas as pl
from jax.experimental.pallas import tpu as pltpu
from jax.experimental.pallas import tpu_sc as plsc

_EMB = 64
_B = 16384
_L = 20
_NC = 2
_NS = 16
_NW = _NC * _NS
_BPW = _B // _NW
_CB = 16
_NSUB = _BPW // _CB
_ROWS = _CB * _L
_DC = 8

_mesh = plsc.VectorSubcoreMesh(
    core_axis_name="c", subcore_axis_name="s", num_cores=_NC, num_subcores=_NS
)


@functools.partial(
    pl.kernel,
    out_type=jax.ShapeDtypeStruct((_B * _L,), jnp.float32),
    mesh=_mesh,
    compiler_params=pltpu.CompilerParams(
        use_tc_tiling_on_sc=True, needs_layout_passes=False),
    scratch_types=[
        pltpu.VMEM((_BPW,), jnp.int32),
        pltpu.VMEM((_BPW * _L,), jnp.int32),
        pltpu.VMEM((2 * _CB, _EMB), jnp.float32),
        pltpu.VMEM((2 * _ROWS, _EMB), jnp.float32),
        pltpu.VMEM((_BPW * _L,), jnp.float32),
        pltpu.SemaphoreType.DMA,
    ],
)
def _sc_skipgram(central_hbm, ctxidx_hbm, win_hbm, wout_hbm, out_hbm,
                 cidx_v, xidx_v, hid_v, ctx_v, out_v, sem):
    wid = lax.axis_index("s") * _NC + lax.axis_index("c")
    base = wid * _BPW
    pltpu.sync_copy(central_hbm.at[pl.ds(base, _BPW)], cidx_v)
    pltpu.sync_copy(ctxidx_hbm.at[pl.ds(base * _L, _BPW * _L)], xidx_v)

    lanes = lax.iota(jnp.int32, 16)

    def fire(sc, hofs, cofs):
        cv = cidx_v[pl.ds(sc * _CB, _CB)]
        for j in range(_CB):
            pltpu.async_copy(win_hbm.at[pl.ds(cv[j], 1)],
                             hid_v.at[pl.ds(hofs + j, 1)], sem)
        for p in range(_ROWS // 16):
            xv = xidx_v[pl.ds(sc * _ROWS + p * 16, 16)]
            for j in range(16):
                pltpu.async_copy(
                    wout_hbm.at[pl.ds(xv[j], 1)],
                    ctx_v.at[pl.ds(cofs + p * 16 + j, 1)], sem)

    fire(0, 0, 0)

    def sub(sc, _):
        p = sc % 2
        hofs = p * _CB
        cofs = p * _ROWS
        pltpu.make_async_copy(
            win_hbm.at[pl.ds(0, _CB)], hid_v.at[pl.ds(hofs, _CB)],
            sem).wait()
        pltpu.make_async_copy(
            wout_hbm.at[pl.ds(0, _ROWS)], ctx_v.at[pl.ds(cofs, _ROWS)],
            sem).wait()

        @pl.when(sc < _NSUB - 1)
        def _():
            q = (sc + 1) % 2
            fire(sc + 1, q * _CB, q * _ROWS)

        hrow = lanes + hofs
        crow = lanes * _L + cofs
        obase = sc * _ROWS + lanes * _L

        def dstep(dc, accs):
            d0 = dc * _DC
            dcols = [jnp.full((16,), d0 + dd, jnp.int32)
                     for dd in range(_DC)]
            hvs = [plsc.load_gather(hid_v, [hrow, dcols[dd]])
                   for dd in range(_DC)]
            out = []
            for l in range(_L):
                rowv = crow + l
                acc = accs[l]
                for dd in range(_DC):
                    cv = plsc.load_gather(ctx_v, [rowv, dcols[dd]])
                    acc = acc + cv * hvs[dd]
                out.append(acc)
            return tuple(out)

        accs = lax.fori_loop(
            0, _EMB // _DC, dstep,
            tuple(jnp.zeros(16, jnp.float32) for _ in range(_L)))

        for l in range(_L):
            sig = 1.0 / (1.0 + jnp.exp(-accs[l]))
            plsc.store_scatter(out_v, [obase + l], sig)
        return 0

    lax.fori_loop(0, _NSUB, sub, 0)
    pltpu.sync_copy(out_v, out_hbm.at[pl.ds(base * _L, _BPW * _L)])


def kernel(central_items, context_items, W_in, W_out):
    out = _sc_skipgram(
        central_items.astype(jnp.int32),
        context_items.reshape(-1).astype(jnp.int32),
        W_in,
        W_out,
    )
    return out.reshape(_B, _L)
